# baseline (speedup 1.0000x reference)
import jax
import jax.numpy as jnp
from jax import lax
from jax.experimental import pallas as pl
from jax.experimental.pallas import tpu as pltpu
from jax.experimental.pallas import tpu_sc as plsc

VOCAB = 100000
EMBED_DIM = 128
N_CTX = 16
B = 1024
L = 200

_NC = 2
_NS = 16
_NW = _NC * _NS
_BPW = B // _NW
_NBUF = 4

_G0_OFF, _G0_LEN = 16, 96
_G1_OFF, _G1_LEN = 112, 88


def _sc_embed(ids_hbm, base_hbm, tok_hbm, out_hbm,
              idx_v, w0, w1, w2, w3, isems, gsems, osems):
  work = (w0, w1, w2, w3)
  wid = lax.axis_index("s") * _NC + lax.axis_index("c")
  base_b = wid * _BPW

  pltpu.sync_copy(ids_hbm.at[pl.ds(base_b * L, _BPW * L)], idx_v)
  for s in range(_NBUF):
    pltpu.sync_copy(base_hbm.at[pl.ds(0, N_CTX)], work[s].at[pl.ds(0, N_CTX)])

  init_d = [None] * _BPW
  gath_d = [None] * _BPW
  out_d = [None] * _BPW

  def start_init(i):
    s = i % _NBUF
    if i >= _NBUF:
      out_d[i - _NBUF].wait()
    init_d[i] = pltpu.async_copy(
        base_hbm.at[pl.ds(N_CTX, L - N_CTX)],
        work[s].at[pl.ds(N_CTX, L - N_CTX)], isems.at[s])

  def start_gather(i):
    s = i % _NBUF
    init_d[i].wait()
    g0 = pltpu.async_copy(
        tok_hbm.at[idx_v.at[pl.ds(i * L + _G0_OFF, _G0_LEN)]],
        work[s].at[pl.ds(_G0_OFF, _G0_LEN)], gsems.at[s], add=True)
    g1 = pltpu.async_copy(
        tok_hbm.at[idx_v.at[pl.ds(i * L + _G1_OFF, _G1_LEN)]],
        work[s].at[pl.ds(_G1_OFF, _G1_LEN)], gsems.at[s], add=True)
    gath_d[i] = (g0, g1)

  def start_out(i):
    s = i % _NBUF
    gath_d[i][0].wait()
    gath_d[i][1].wait()
    out_d[i] = pltpu.async_copy(work[s], out_hbm.at[base_b + i], osems.at[s])

  for step in range(_BPW + 2):
    if step < _BPW:
      start_init(step)
    if 1 <= step and step - 1 < _BPW:
      start_gather(step - 1)
    if 2 <= step and step - 2 < _BPW:
      start_out(step - 2)
  for i in range(_BPW - _NBUF, _BPW):
    out_d[i].wait()


@jax.jit
def kernel(input_ids, token_table, position_table, ctx):
  ids = input_ids.astype(jnp.int32).reshape(-1)
  prefix = ctx[:N_CTX] + position_table[:N_CTX]
  base = jnp.concatenate([prefix, position_table[N_CTX:L]], axis=0)

  mesh = plsc.VectorSubcoreMesh(core_axis_name="c", subcore_axis_name="s")
  run = pl.kernel(
      _sc_embed,
      out_type=jax.ShapeDtypeStruct((B, L, EMBED_DIM), jnp.float32),
      mesh=mesh,
      scratch_types=[
          pltpu.VMEM((_BPW * L,), jnp.int32),
          pltpu.VMEM((L, EMBED_DIM), jnp.float32),
          pltpu.VMEM((L, EMBED_DIM), jnp.float32),
          pltpu.VMEM((L, EMBED_DIM), jnp.float32),
          pltpu.VMEM((L, EMBED_DIM), jnp.float32),
          pltpu.SemaphoreType.DMA((_NBUF,)),
          pltpu.SemaphoreType.DMA((_NBUF,)),
          pltpu.SemaphoreType.DMA((_NBUF,)),
      ],
  )
  return run(ids, base, token_table)


# speedup vs baseline: 1.8687x; 1.8687x over previous
import jax
import jax.numpy as jnp
from jax import lax
from jax.experimental import pallas as pl
from jax.experimental.pallas import tpu as pltpu
from jax.experimental.pallas import tpu_sc as plsc

VOCAB = 100000
EMBED_DIM = 128
N_CTX = 16
B = 1024
L = 200

_NC = 2
_NS = 16
_NW = _NC * _NS
_BPW = B // _NW

_G0_OFF, _G0_LEN = 16, 96
_G1_OFF, _G1_LEN = 112, 88


def _sc_embed(ids_hbm, base_hbm, tok_hbm, out_hbm,
              idx_v, work_v, base_sh, sem):
  wid = lax.axis_index("s") * _NC + lax.axis_index("c")
  base_b = wid * _BPW

  @pl.when(lax.axis_index("s") == 0)
  def _():
    pltpu.sync_copy(base_hbm, base_sh)

  pltpu.sync_copy(ids_hbm.at[pl.ds(base_b * L, _BPW * L)], idx_v)
  plsc.subcore_barrier()

  pltpu.sync_copy(base_sh.at[pl.ds(0, N_CTX)], work_v.at[pl.ds(0, N_CTX)])

  def body(i, carry):
    pltpu.sync_copy(base_sh.at[pl.ds(N_CTX, L - N_CTX)],
                    work_v.at[pl.ds(N_CTX, L - N_CTX)])

    cp0 = pltpu.async_copy(
        tok_hbm.at[idx_v.at[pl.ds(i * L + _G0_OFF, _G0_LEN)]],
        work_v.at[pl.ds(_G0_OFF, _G0_LEN)], sem, add=True)
    cp1 = pltpu.async_copy(
        tok_hbm.at[idx_v.at[pl.ds(i * L + _G1_OFF, _G1_LEN)]],
        work_v.at[pl.ds(_G1_OFF, _G1_LEN)], sem, add=True)
    cp0.wait()
    cp1.wait()

    pltpu.sync_copy(work_v, out_hbm.at[base_b + i])
    return carry

  lax.fori_loop(0, _BPW, body, 0)


@jax.jit
def kernel(input_ids, token_table, position_table, ctx):
  ids = input_ids.astype(jnp.int32).reshape(-1)
  prefix = ctx[:N_CTX] + position_table[:N_CTX]
  base = jnp.concatenate([prefix, position_table[N_CTX:L]], axis=0)

  mesh = plsc.VectorSubcoreMesh(core_axis_name="c", subcore_axis_name="s")
  run = pl.kernel(
      _sc_embed,
      out_type=jax.ShapeDtypeStruct((B, L, EMBED_DIM), jnp.float32),
      mesh=mesh,
      scratch_types=[
          pltpu.VMEM((_BPW * L,), jnp.int32),
          pltpu.VMEM((L, EMBED_DIM), jnp.float32),
          pltpu.VMEM_SHARED((L, EMBED_DIM), jnp.float32),
          pltpu.SemaphoreType.DMA,
      ],
  )
  return run(ids, base, token_table)


# speedup vs baseline: 2.6307x; 1.4077x over previous
import jax
import jax.numpy as jnp
from jax import lax
from jax.experimental import pallas as pl
from jax.experimental.pallas import tpu as pltpu
from jax.experimental.pallas import tpu_sc as plsc

VOCAB = 100000
EMBED_DIM = 128
N_CTX = 16
B = 1024
L = 200

_NC = 2
_NS = 16
_NW = _NC * _NS
_BPW = B // _NW

_G0_OFF, _G0_LEN = 16, 96
_G1_OFF, _G1_LEN = 112, 88


def _sc_embed(ids_hbm, base_hbm, tok_hbm, out_hbm,
              idx_v, w0, w1, base_sh, gsems, osems):
  work = (w0, w1)
  wid = lax.axis_index("s") * _NC + lax.axis_index("c")
  base_b = wid * _BPW

  @pl.when(lax.axis_index("s") == 0)
  def _():
    pltpu.sync_copy(base_hbm, base_sh)

  pltpu.sync_copy(ids_hbm.at[pl.ds(base_b * L, _BPW * L)], idx_v)
  plsc.subcore_barrier()

  for s in range(2):
    pltpu.sync_copy(base_sh.at[pl.ds(0, N_CTX)], work[s].at[pl.ds(0, N_CTX)])

  gath_d = [None] * _BPW
  out_d = [None] * _BPW
  for step in range(_BPW + 1):
    if step < _BPW:
      s = step % 2
      if step >= 2:
        out_d[step - 2].wait()
      pltpu.sync_copy(base_sh.at[pl.ds(N_CTX, L - N_CTX)],
                      work[s].at[pl.ds(N_CTX, L - N_CTX)])
      g0 = pltpu.async_copy(
          tok_hbm.at[idx_v.at[pl.ds(step * L + _G0_OFF, _G0_LEN)]],
          work[s].at[pl.ds(_G0_OFF, _G0_LEN)], gsems.at[s], add=True)
      g1 = pltpu.async_copy(
          tok_hbm.at[idx_v.at[pl.ds(step * L + _G1_OFF, _G1_LEN)]],
          work[s].at[pl.ds(_G1_OFF, _G1_LEN)], gsems.at[s], add=True)
      gath_d[step] = (g0, g1)
    if step >= 1:
      j = step - 1
      s = j % 2
      gath_d[j][0].wait()
      gath_d[j][1].wait()
      out_d[j] = pltpu.async_copy(work[s], out_hbm.at[base_b + j],
                                  osems.at[s])
  out_d[_BPW - 2].wait()
  out_d[_BPW - 1].wait()


@jax.jit
def kernel(input_ids, token_table, position_table, ctx):
  ids = input_ids.astype(jnp.int32).reshape(-1)
  prefix = ctx[:N_CTX] + position_table[:N_CTX]
  base = jnp.concatenate([prefix, position_table[N_CTX:L]], axis=0)

  mesh = plsc.VectorSubcoreMesh(core_axis_name="c", subcore_axis_name="s")
  run = pl.kernel(
      _sc_embed,
      out_type=jax.ShapeDtypeStruct((B, L, EMBED_DIM), jnp.float32),
      mesh=mesh,
      scratch_types=[
          pltpu.VMEM((_BPW * L,), jnp.int32),
          pltpu.VMEM((L, EMBED_DIM), jnp.float32),
          pltpu.VMEM((L, EMBED_DIM), jnp.float32),
          pltpu.VMEM_SHARED((L, EMBED_DIM), jnp.float32),
          pltpu.SemaphoreType.DMA((2,)),
          pltpu.SemaphoreType.DMA((2,)),
      ],
  )
  return run(ids, base, token_table)


# speedup vs baseline: 2.7037x; 1.0278x over previous
import jax
import jax.numpy as jnp
from jax import lax
from jax.experimental import pallas as pl
from jax.experimental.pallas import tpu as pltpu
from jax.experimental.pallas import tpu_sc as plsc

VOCAB = 100000
EMBED_DIM = 128
N_CTX = 16
B = 1024
L = 200

_NC = 2
_NS = 16
_NW = _NC * _NS
_BPW = B // _NW

_G0_OFF, _G0_LEN = 16, 96
_G1_OFF, _G1_LEN = 112, 88


def _sc_embed(ids_hbm, base_hbm, tok_hbm, out_hbm,
              idx_v, w0, w1, w2, base_sh, isems, gsems, osems):
  work = (w0, w1, w2)
  nbuf = len(work)
  wid = lax.axis_index("s") * _NC + lax.axis_index("c")
  base_b = wid * _BPW

  @pl.when(lax.axis_index("s") == 0)
  def _():
    pltpu.sync_copy(base_hbm, base_sh)

  pltpu.sync_copy(ids_hbm.at[pl.ds(base_b * L, _BPW * L)], idx_v)
  plsc.subcore_barrier()

  for s in range(nbuf):
    pltpu.sync_copy(base_sh.at[pl.ds(0, N_CTX)], work[s].at[pl.ds(0, N_CTX)])

  init_d = [None] * _BPW
  gath_d = [None] * _BPW
  out_d = [None] * _BPW
  for step in range(_BPW + 2):
    if step < _BPW:
      s = step % nbuf
      if step >= nbuf:
        out_d[step - nbuf].wait()
      init_d[step] = pltpu.async_copy(
          base_sh.at[pl.ds(N_CTX, L - N_CTX)],
          work[s].at[pl.ds(N_CTX, L - N_CTX)], isems.at[s])
    if 1 <= step and step - 1 < _BPW:
      j = step - 1
      s = j % nbuf
      init_d[j].wait()
      g0 = pltpu.async_copy(
          tok_hbm.at[idx_v.at[pl.ds(j * L + _G0_OFF, _G0_LEN)]],
          work[s].at[pl.ds(_G0_OFF, _G0_LEN)], gsems.at[s], add=True)
      g1 = pltpu.async_copy(
          tok_hbm.at[idx_v.at[pl.ds(j * L + _G1_OFF, _G1_LEN)]],
          work[s].at[pl.ds(_G1_OFF, _G1_LEN)], gsems.at[s], add=True)
      gath_d[j] = (g0, g1)
    if 2 <= step and step - 2 < _BPW:
      j = step - 2
      s = j % nbuf
      gath_d[j][0].wait()
      gath_d[j][1].wait()
      out_d[j] = pltpu.async_copy(work[s], out_hbm.at[base_b + j],
                                  osems.at[s])
  for j in range(_BPW - nbuf, _BPW):
    out_d[j].wait()


@jax.jit
def kernel(input_ids, token_table, position_table, ctx):
  ids = input_ids.astype(jnp.int32).reshape(-1)
  prefix = ctx[:N_CTX] + position_table[:N_CTX]
  base = jnp.concatenate([prefix, position_table[N_CTX:L]], axis=0)

  mesh = plsc.VectorSubcoreMesh(core_axis_name="c", subcore_axis_name="s")
  run = pl.kernel(
      _sc_embed,
      out_type=jax.ShapeDtypeStruct((B, L, EMBED_DIM), jnp.float32),
      mesh=mesh,
      scratch_types=[
          pltpu.VMEM((_BPW * L,), jnp.int32),
          pltpu.VMEM((L, EMBED_DIM), jnp.float32),
          pltpu.VMEM((L, EMBED_DIM), jnp.float32),
          pltpu.VMEM((L, EMBED_DIM), jnp.float32),
          pltpu.VMEM_SHARED((L, EMBED_DIM), jnp.float32),
          pltpu.SemaphoreType.DMA((3,)),
          pltpu.SemaphoreType.DMA((3,)),
          pltpu.SemaphoreType.DMA((3,)),
      ],
  )
  return run(ids, base, token_table)
